# baseline (speedup 1.0000x reference)
import functools

import jax
import jax.numpy as jnp
import numpy as np
from jax import lax
from jax.experimental import pallas as pl
from jax.experimental.pallas import tpu as pltpu
from jax.experimental.pallas import tpu_sc as plsc

_K = 16


def _proj_knn_body(xyz_ref, xyzT_ref, feat_ref, W1_ref, b1_ref, Wq_ref,
                   Wk_ref, Wv_ref, q_ref, kv_ref, idx_ref):
    b = pl.program_id(0)
    n_full = xyzT_ref.shape[2]

    f = feat_ref[0]
    x = jnp.dot(f, W1_ref[...], preferred_element_type=jnp.float32)
    x = x + b1_ref[...]
    q_ref[0] = jnp.dot(x, Wq_ref[...], preferred_element_type=jnp.float32)
    kx = jnp.dot(x, Wk_ref[...], preferred_element_type=jnp.float32)
    vx = jnp.dot(x, Wv_ref[...], preferred_element_type=jnp.float32)
    kb = lax.bitcast_convert_type(kx.astype(jnp.bfloat16), jnp.uint16)
    vb = lax.bitcast_convert_type(vx.astype(jnp.bfloat16), jnp.uint16)
    packed = kb.astype(jnp.uint32) | (vb.astype(jnp.uint32) << 16)
    kv_ref[0] = lax.bitcast_convert_type(packed, jnp.float32)

    xyzb = xyz_ref[0]
    xyzT = xyzT_ref[0]
    dot = lax.dot_general(xyzb, xyzT, (((1,), (0,)), ((), ())),
                          preferred_element_type=jnp.float32)
    si = jnp.sum(xyzb * xyzb, axis=1, keepdims=True)
    sj = jnp.sum(xyzT * xyzT, axis=0, keepdims=True)
    d = (-2.0 * dot) + si
    d = d + sj

    cols = []
    for _ in range(_K):
        amin = jnp.argmin(d, axis=1).astype(jnp.int32)[:, None]
        cols.append(amin)
        col = lax.broadcasted_iota(jnp.int32, d.shape, 1)
        d = jnp.where(col == amin, jnp.float32(np.inf), d)
    idxblk = jnp.concatenate(cols, axis=1)
    idx_ref[0] = idxblk + b * n_full


def _proj_knn(xyz, xyzT, features, W1, b1, Wq, Wk, Wv):
    B, N, D = features.shape
    P1 = 256
    full = lambda shape: pl.BlockSpec(shape, lambda b, i: (0, 0))
    return pl.pallas_call(
        _proj_knn_body,
        grid=(B, N // P1),
        in_specs=[
            pl.BlockSpec((1, P1, 3), lambda b, i: (b, i, 0)),
            pl.BlockSpec((1, 3, N), lambda b, i: (b, 0, 0)),
            pl.BlockSpec((1, P1, D), lambda b, i: (b, i, 0)),
            full((D, D)),
            full((1, D)),
            full((D, D)),
            full((D, D)),
            full((D, D)),
        ],
        out_specs=[
            pl.BlockSpec((1, P1, D), lambda b, i: (b, i, 0)),
            pl.BlockSpec((1, P1, D), lambda b, i: (b, i, 0)),
            pl.BlockSpec((1, P1, _K), lambda b, i: (b, i, 0)),
        ],
        out_shape=[
            jax.ShapeDtypeStruct((B, N, D), jnp.float32),
            jax.ShapeDtypeStruct((B, N, D), jnp.float32),
            jax.ShapeDtypeStruct((B, N, _K), jnp.int32),
        ],
    )(xyz, xyzT, features, W1, b1, Wq, Wk, Wv)


def _sc_gather(table_kv, table_x, flat_idx):
    TOT = flat_idx.shape[0]
    w_kv = table_kv.shape[1]
    w_x = table_x.shape[1]
    info = plsc.get_sparse_core_info()
    NC, NS = info.num_cores, info.num_subcores
    NW = NC * NS
    per_w = TOT // NW
    CH = 128
    n_ch = per_w // CH
    mesh = plsc.VectorSubcoreMesh(core_axis_name="c", subcore_axis_name="s")

    @functools.partial(
        pl.kernel,
        mesh=mesh,
        out_type=[
            jax.ShapeDtypeStruct((TOT, w_kv), jnp.float32),
            jax.ShapeDtypeStruct((TOT, w_x), jnp.float32),
        ],
        scratch_types=[
            pltpu.VMEM((CH,), jnp.int32),
            pltpu.VMEM((CH,), jnp.int32),
            pltpu.VMEM((CH, w_kv), jnp.float32),
            pltpu.VMEM((CH, w_x), jnp.float32),
            pltpu.VMEM((CH, w_kv), jnp.float32),
            pltpu.VMEM((CH, w_x), jnp.float32),
            pltpu.SemaphoreType.DMA,
            pltpu.SemaphoreType.DMA,
            pltpu.SemaphoreType.DMA,
            pltpu.SemaphoreType.DMA,
        ],
    )
    def gath(kv_hbm, x_hbm, idx_hbm, okv_hbm, ox_hbm, idx_a, idx_b, rkv_a,
             rx_a, rkv_b, rx_b, sem1a, sem2a, sem1b, sem2b):
        wid = lax.axis_index("s") * NC + lax.axis_index("c")
        base = wid * per_w

        def body(j, carry):
            offa = base + (2 * j) * CH
            offb = offa + CH
            pltpu.sync_copy(idx_hbm.at[pl.ds(offa, CH)], idx_a)
            c1a = pltpu.async_copy(kv_hbm.at[idx_a], rkv_a, sem1a)
            c2a = pltpu.async_copy(x_hbm.at[idx_a], rx_a, sem2a)
            pltpu.sync_copy(idx_hbm.at[pl.ds(offb, CH)], idx_b)
            c1b = pltpu.async_copy(kv_hbm.at[idx_b], rkv_b, sem1b)
            c2b = pltpu.async_copy(x_hbm.at[idx_b], rx_b, sem2b)
            c1a.wait()
            c2a.wait()
            pltpu.sync_copy(rkv_a, okv_hbm.at[pl.ds(offa, CH)])
            pltpu.sync_copy(rx_a, ox_hbm.at[pl.ds(offa, CH)])
            c1b.wait()
            c2b.wait()
            pltpu.sync_copy(rkv_b, okv_hbm.at[pl.ds(offb, CH)])
            pltpu.sync_copy(rx_b, ox_hbm.at[pl.ds(offb, CH)])
            return carry

        lax.fori_loop(0, n_ch // 2, body, 0)

    return gath(table_kv, table_x, flat_idx)


def _attn_body(q_ref, feat_ref, xyzp_ref, gkv_ref, gx_ref, Wd1_ref, bd1_ref,
               Wd2_ref, bd2_ref, Wg1_ref, bg1_ref, Wg2_ref, bg2_ref, W2_ref,
               b2_ref, nf_ref, attn_ref):
    P2, K, D = gkv_ref.shape
    gkv = gkv_ref[...]
    u = lax.bitcast_convert_type(gkv, jnp.int32)
    gk = lax.bitcast_convert_type(u << 16, jnp.float32).reshape(P2 * K, D)
    gv = lax.bitcast_convert_type(u & jnp.int32(-65536), jnp.float32)
    gx = gx_ref[:, :, :16]

    xyzp = xyzp_ref[...]
    delta = (xyzp[:, None, :] - gx).reshape(P2 * K, 16)
    h1 = jnp.dot(delta.astype(jnp.bfloat16), Wd1_ref[...],
                 preferred_element_type=jnp.float32)
    h1 = jnp.maximum(h1 + bd1_ref[...], 0.0)
    pos = jnp.dot(h1.astype(jnp.bfloat16), Wd2_ref[...],
                  preferred_element_type=jnp.float32)
    pos = pos + bd2_ref[...]

    q = q_ref[...]
    qb = jnp.broadcast_to(q[:, None, :], (P2, K, D)).reshape(P2 * K, D)
    pre = qb - gk + pos
    h2 = jnp.dot(pre.astype(jnp.bfloat16), Wg1_ref[...],
                 preferred_element_type=jnp.float32)
    h2 = jnp.maximum(h2 + bg1_ref[...], 0.0)
    logits = jnp.dot(h2.astype(jnp.bfloat16), Wg2_ref[...],
                     preferred_element_type=jnp.float32)
    logits = (logits + bg2_ref[...]) * (1.0 / 16.0)

    l3 = logits.reshape(P2, K, D)
    m = jnp.max(l3, axis=1, keepdims=True)
    e = jnp.exp(l3 - m)
    s = jnp.sum(e, axis=1, keepdims=True)
    attn = e / s
    attn_ref[...] = attn

    wsum = jnp.sum(attn * (gv + pos.reshape(P2, K, D)), axis=1)
    nf = jnp.dot(wsum.astype(jnp.bfloat16), W2_ref[...],
                 preferred_element_type=jnp.float32)
    nf_ref[...] = nf + b2_ref[...] + feat_ref[...]


def _attn(q, feat, xyzp, gkv, gx, Wd1p, bd1, Wd2, bd2, Wg1, bg1, Wg2, bg2,
          W2, b2):
    BN, D = q.shape
    P2 = 128
    full = lambda shape: pl.BlockSpec(shape, lambda i: (0, 0))
    return pl.pallas_call(
        _attn_body,
        grid=(BN // P2,),
        in_specs=[
            pl.BlockSpec((P2, D), lambda i: (i, 0)),
            pl.BlockSpec((P2, D), lambda i: (i, 0)),
            pl.BlockSpec((P2, 16), lambda i: (i, 0)),
            pl.BlockSpec((P2, _K, D), lambda i: (i, 0, 0)),
            pl.BlockSpec((P2, _K, 128), lambda i: (i, 0, 0)),
            full((16, D)),
            full((1, D)),
            full((D, D)),
            full((1, D)),
            full((D, D)),
            full((1, D)),
            full((D, D)),
            full((1, D)),
            full((D, D)),
            full((1, D)),
        ],
        out_specs=[
            pl.BlockSpec((P2, D), lambda i: (i, 0)),
            pl.BlockSpec((P2, _K, D), lambda i: (i, 0, 0)),
        ],
        out_shape=[
            jax.ShapeDtypeStruct((BN, D), jnp.float32),
            jax.ShapeDtypeStruct((BN, _K, D), jnp.float32),
        ],
    )(q, feat, xyzp, gkv, gx, Wd1p, bd1, Wd2, bd2, Wg1, bg1, Wg2, bg2, W2, b2)


def _half_pipeline(xyz, features, W1, b1, Wd1p, bd1, Wd2, bd2, Wg1, bg1,
                   Wg2, bg2, W2, b2, Wq, Wk, Wv):
    B, N, _ = xyz.shape
    D = features.shape[-1]
    BN = B * N

    xyzT = jnp.swapaxes(xyz, 1, 2)
    q, kv, gidx = _proj_knn(
        xyz, xyzT, features, W1, b1, Wq, Wk, Wv)

    xyzp = jnp.concatenate(
        [xyz, jnp.zeros((B, N, 13), jnp.float32)], axis=-1)
    xyzp128 = jnp.concatenate(
        [xyzp, jnp.zeros((B, N, 112), jnp.float32)], axis=-1)
    flat_idx = gidx.reshape(BN * _K)

    gkv, gx = _sc_gather(kv.reshape(BN, D), xyzp128.reshape(BN, 128),
                         flat_idx)
    gkv = gkv.reshape(BN, _K, D)
    gx = gx.reshape(BN, _K, 128)

    nf, attn = _attn(
        q.reshape(BN, D), features.reshape(BN, D), xyzp.reshape(BN, 16),
        gkv, gx, Wd1p, bd1, Wd2, bd2, Wg1, bg1, Wg2, bg2, W2, b2)

    return (nf.reshape(B, N, D), attn.reshape(B, N, _K, D))


def kernel(xyz, features, W1, b1, W2, b2, Wd1, bd1, Wd2, bd2, Wg1, bg1,
           Wg2, bg2, Wq, Wk, Wv):
    B, N, _ = xyz.shape
    D = features.shape[-1]

    Wd1p = jnp.concatenate(
        [Wd1, jnp.zeros((13, D), jnp.float32)], axis=0)
    bf = jnp.bfloat16
    ws = (W1, b1.reshape(1, D), Wd1p.astype(bf), bd1.reshape(1, D), Wd2.astype(bf),
          bd2.reshape(1, D), Wg1.astype(bf), bg1.reshape(1, D),
          Wg2.astype(bf), bg2.reshape(1, D), W2.astype(bf), b2.reshape(1, D),
          Wq, Wk, Wv)

    outs = []
    for b in range(B):
        s = slice(b, b + 1)
        outs.append(_half_pipeline(xyz[s], features[s], *ws))
    nf = jnp.concatenate([o[0] for o in outs], axis=0)
    attn = jnp.concatenate([o[1] for o in outs], axis=0)
    return (nf, attn)


# speedup vs baseline: 1.0155x; 1.0155x over previous
import functools

import jax
import jax.numpy as jnp
import numpy as np
from jax import lax
from jax.experimental import pallas as pl
from jax.experimental.pallas import tpu as pltpu
from jax.experimental.pallas import tpu_sc as plsc

_K = 16


def _proj_knn_body(xyz_ref, xyzT_ref, feat_ref, W1_ref, b1_ref, Wq_ref,
                   Wk_ref, Wv_ref, q_ref, kv_ref, idx_ref):
    b = pl.program_id(0)
    n_full = xyzT_ref.shape[2]

    f = feat_ref[0]
    x = jnp.dot(f.astype(jnp.bfloat16), W1_ref[...],
                preferred_element_type=jnp.float32)
    x = x + b1_ref[...]
    xb = x.astype(jnp.bfloat16)
    q_ref[0] = jnp.dot(xb, Wq_ref[...], preferred_element_type=jnp.float32)
    kx = jnp.dot(xb, Wk_ref[...], preferred_element_type=jnp.float32)
    vx = jnp.dot(xb, Wv_ref[...], preferred_element_type=jnp.float32)
    kb = lax.bitcast_convert_type(kx.astype(jnp.bfloat16), jnp.uint16)
    vb = lax.bitcast_convert_type(vx.astype(jnp.bfloat16), jnp.uint16)
    packed = kb.astype(jnp.uint32) | (vb.astype(jnp.uint32) << 16)
    kv_ref[0] = lax.bitcast_convert_type(packed, jnp.float32)

    xyzb = xyz_ref[0]
    xyzT = xyzT_ref[0]
    dot = lax.dot_general(xyzb, xyzT, (((1,), (0,)), ((), ())),
                          preferred_element_type=jnp.float32)
    si = jnp.sum(xyzb * xyzb, axis=1, keepdims=True)
    sj = jnp.sum(xyzT * xyzT, axis=0, keepdims=True)
    d = (-2.0 * dot) + si
    d = d + sj

    cols = []
    for _ in range(_K):
        amin = jnp.argmin(d, axis=1).astype(jnp.int32)[:, None]
        cols.append(amin)
        col = lax.broadcasted_iota(jnp.int32, d.shape, 1)
        d = jnp.where(col == amin, jnp.float32(np.inf), d)
    idxblk = jnp.concatenate(cols, axis=1)
    idx_ref[0] = idxblk + b * n_full


def _proj_knn(xyz, xyzT, features, W1, b1, Wq, Wk, Wv):
    B, N, D = features.shape
    P1 = 256
    full = lambda shape: pl.BlockSpec(shape, lambda b, i: (0, 0))
    return pl.pallas_call(
        _proj_knn_body,
        grid=(B, N // P1),
        in_specs=[
            pl.BlockSpec((1, P1, 3), lambda b, i: (b, i, 0)),
            pl.BlockSpec((1, 3, N), lambda b, i: (b, 0, 0)),
            pl.BlockSpec((1, P1, D), lambda b, i: (b, i, 0)),
            full((D, D)),
            full((1, D)),
            full((D, D)),
            full((D, D)),
            full((D, D)),
        ],
        out_specs=[
            pl.BlockSpec((1, P1, D), lambda b, i: (b, i, 0)),
            pl.BlockSpec((1, P1, D), lambda b, i: (b, i, 0)),
            pl.BlockSpec((1, P1, _K), lambda b, i: (b, i, 0)),
        ],
        out_shape=[
            jax.ShapeDtypeStruct((B, N, D), jnp.float32),
            jax.ShapeDtypeStruct((B, N, D), jnp.float32),
            jax.ShapeDtypeStruct((B, N, _K), jnp.int32),
        ],
    )(xyz, xyzT, features, W1, b1, Wq, Wk, Wv)


def _sc_gather(table_kv, table_x, flat_idx):
    TOT = flat_idx.shape[0]
    w_kv = table_kv.shape[1]
    w_x = table_x.shape[1]
    info = plsc.get_sparse_core_info()
    NC, NS = info.num_cores, info.num_subcores
    NW = NC * NS
    per_w = TOT // NW
    CH = 128
    n_ch = per_w // CH
    mesh = plsc.VectorSubcoreMesh(core_axis_name="c", subcore_axis_name="s")

    @functools.partial(
        pl.kernel,
        mesh=mesh,
        out_type=[
            jax.ShapeDtypeStruct((TOT, w_kv), jnp.float32),
            jax.ShapeDtypeStruct((TOT, w_x), jnp.float32),
        ],
        scratch_types=[
            pltpu.VMEM((CH,), jnp.int32),
            pltpu.VMEM((CH,), jnp.int32),
            pltpu.VMEM((CH, w_kv), jnp.float32),
            pltpu.VMEM((CH, w_x), jnp.float32),
            pltpu.VMEM((CH, w_kv), jnp.float32),
            pltpu.VMEM((CH, w_x), jnp.float32),
            pltpu.SemaphoreType.DMA,
            pltpu.SemaphoreType.DMA,
            pltpu.SemaphoreType.DMA,
            pltpu.SemaphoreType.DMA,
        ],
    )
    def gath(kv_hbm, x_hbm, idx_hbm, okv_hbm, ox_hbm, idx_a, idx_b, rkv_a,
             rx_a, rkv_b, rx_b, sem1a, sem2a, sem1b, sem2b):
        wid = lax.axis_index("s") * NC + lax.axis_index("c")
        base = wid * per_w

        def body(j, carry):
            offa = base + (2 * j) * CH
            offb = offa + CH
            pltpu.sync_copy(idx_hbm.at[pl.ds(offa, CH)], idx_a)
            c1a = pltpu.async_copy(kv_hbm.at[idx_a], rkv_a, sem1a)
            c2a = pltpu.async_copy(x_hbm.at[idx_a], rx_a, sem2a)
            pltpu.sync_copy(idx_hbm.at[pl.ds(offb, CH)], idx_b)
            c1b = pltpu.async_copy(kv_hbm.at[idx_b], rkv_b, sem1b)
            c2b = pltpu.async_copy(x_hbm.at[idx_b], rx_b, sem2b)
            c1a.wait()
            c2a.wait()
            pltpu.sync_copy(rkv_a, okv_hbm.at[pl.ds(offa, CH)])
            pltpu.sync_copy(rx_a, ox_hbm.at[pl.ds(offa, CH)])
            c1b.wait()
            c2b.wait()
            pltpu.sync_copy(rkv_b, okv_hbm.at[pl.ds(offb, CH)])
            pltpu.sync_copy(rx_b, ox_hbm.at[pl.ds(offb, CH)])
            return carry

        lax.fori_loop(0, n_ch // 2, body, 0)

    return gath(table_kv, table_x, flat_idx)


def _attn_body(q_ref, feat_ref, xyzp_ref, gkv_ref, gx_ref, Wd1_ref, bd1_ref,
               Wd2_ref, bd2_ref, Wg1_ref, bg1_ref, Wg2_ref, bg2_ref, W2_ref,
               b2_ref, nf_ref, attn_ref):
    P2, K, D = gkv_ref.shape
    gkv = gkv_ref[...]
    u = lax.bitcast_convert_type(gkv, jnp.int32)
    gk = lax.bitcast_convert_type(u << 16, jnp.float32).reshape(P2 * K, D)
    gv = lax.bitcast_convert_type(u & jnp.int32(-65536), jnp.float32)
    gx = gx_ref[:, :, :16]

    xyzp = xyzp_ref[...]
    delta = (xyzp[:, None, :] - gx).reshape(P2 * K, 16)
    h1 = jnp.dot(delta.astype(jnp.bfloat16), Wd1_ref[...],
                 preferred_element_type=jnp.float32)
    h1 = jnp.maximum(h1 + bd1_ref[...], 0.0)
    pos = jnp.dot(h1.astype(jnp.bfloat16), Wd2_ref[...],
                  preferred_element_type=jnp.float32)
    pos = pos + bd2_ref[...]

    q = q_ref[...]
    qb = jnp.broadcast_to(q[:, None, :], (P2, K, D)).reshape(P2 * K, D)
    pre = qb - gk + pos
    h2 = jnp.dot(pre.astype(jnp.bfloat16), Wg1_ref[...],
                 preferred_element_type=jnp.float32)
    h2 = jnp.maximum(h2 + bg1_ref[...], 0.0)
    logits = jnp.dot(h2.astype(jnp.bfloat16), Wg2_ref[...],
                     preferred_element_type=jnp.float32)
    logits = (logits + bg2_ref[...]) * (1.0 / 16.0)

    l3 = logits.reshape(P2, K, D)
    m = jnp.max(l3, axis=1, keepdims=True)
    e = jnp.exp(l3 - m)
    s = jnp.sum(e, axis=1, keepdims=True)
    attn = e / s
    attn_ref[...] = attn

    wsum = jnp.sum(attn * (gv + pos.reshape(P2, K, D)), axis=1)
    nf = jnp.dot(wsum.astype(jnp.bfloat16), W2_ref[...],
                 preferred_element_type=jnp.float32)
    nf_ref[...] = nf + b2_ref[...] + feat_ref[...]


def _attn(q, feat, xyzp, gkv, gx, Wd1p, bd1, Wd2, bd2, Wg1, bg1, Wg2, bg2,
          W2, b2):
    BN, D = q.shape
    P2 = 256
    full = lambda shape: pl.BlockSpec(shape, lambda i: (0, 0))
    return pl.pallas_call(
        _attn_body,
        grid=(BN // P2,),
        in_specs=[
            pl.BlockSpec((P2, D), lambda i: (i, 0)),
            pl.BlockSpec((P2, D), lambda i: (i, 0)),
            pl.BlockSpec((P2, 16), lambda i: (i, 0)),
            pl.BlockSpec((P2, _K, D), lambda i: (i, 0, 0)),
            pl.BlockSpec((P2, _K, 128), lambda i: (i, 0, 0)),
            full((16, D)),
            full((1, D)),
            full((D, D)),
            full((1, D)),
            full((D, D)),
            full((1, D)),
            full((D, D)),
            full((1, D)),
            full((D, D)),
            full((1, D)),
        ],
        out_specs=[
            pl.BlockSpec((P2, D), lambda i: (i, 0)),
            pl.BlockSpec((P2, _K, D), lambda i: (i, 0, 0)),
        ],
        out_shape=[
            jax.ShapeDtypeStruct((BN, D), jnp.float32),
            jax.ShapeDtypeStruct((BN, _K, D), jnp.float32),
        ],
    )(q, feat, xyzp, gkv, gx, Wd1p, bd1, Wd2, bd2, Wg1, bg1, Wg2, bg2, W2, b2)


def _half_pipeline(xyz, features, W1, b1, Wd1p, bd1, Wd2, bd2, Wg1, bg1,
                   Wg2, bg2, W2, b2, Wq, Wk, Wv):
    B, N, _ = xyz.shape
    D = features.shape[-1]
    BN = B * N

    xyzT = jnp.swapaxes(xyz, 1, 2)
    q, kv, gidx = _proj_knn(
        xyz, xyzT, features, W1, b1, Wq, Wk, Wv)

    xyzp = jnp.concatenate(
        [xyz, jnp.zeros((B, N, 13), jnp.float32)], axis=-1)
    xyzp128 = jnp.concatenate(
        [xyzp, jnp.zeros((B, N, 112), jnp.float32)], axis=-1)
    flat_idx = gidx.reshape(BN * _K)

    gkv, gx = _sc_gather(kv.reshape(BN, D), xyzp128.reshape(BN, 128),
                         flat_idx)
    gkv = gkv.reshape(BN, _K, D)
    gx = gx.reshape(BN, _K, 128)

    nf, attn = _attn(
        q.reshape(BN, D), features.reshape(BN, D), xyzp.reshape(BN, 16),
        gkv, gx, Wd1p, bd1, Wd2, bd2, Wg1, bg1, Wg2, bg2, W2, b2)

    return (nf.reshape(B, N, D), attn.reshape(B, N, _K, D))


def kernel(xyz, features, W1, b1, W2, b2, Wd1, bd1, Wd2, bd2, Wg1, bg1,
           Wg2, bg2, Wq, Wk, Wv):
    B, N, _ = xyz.shape
    D = features.shape[-1]

    Wd1p = jnp.concatenate(
        [Wd1, jnp.zeros((13, D), jnp.float32)], axis=0)
    bf = jnp.bfloat16
    ws = (W1.astype(bf), b1.reshape(1, D), Wd1p.astype(bf),
          bd1.reshape(1, D), Wd2.astype(bf), bd2.reshape(1, D),
          Wg1.astype(bf), bg1.reshape(1, D), Wg2.astype(bf),
          bg2.reshape(1, D), W2.astype(bf), b2.reshape(1, D),
          Wq.astype(bf), Wk.astype(bf), Wv.astype(bf))

    h = B // 2
    outs = []
    for s in (slice(0, h), slice(h, B)):
        outs.append(_half_pipeline(xyz[s], features[s], *ws))
    nf = jnp.concatenate([o[0] for o in outs], axis=0)
    attn = jnp.concatenate([o[1] for o in outs], axis=0)
    return (nf, attn)


# speedup vs baseline: 1.0448x; 1.0289x over previous
import functools

import jax
import jax.numpy as jnp
import numpy as np
from jax import lax
from jax.experimental import pallas as pl
from jax.experimental.pallas import tpu as pltpu
from jax.experimental.pallas import tpu_sc as plsc

_K = 16


def _proj_knn_body(xyz_ref, xyzT_ref, feat_ref, W1_ref, b1_ref, Wq_ref,
                   Wk_ref, Wv_ref, q_ref, kv_ref, idx_ref):
    b = pl.program_id(0)
    n_full = xyzT_ref.shape[2]

    f = feat_ref[0]
    x = jnp.dot(f.astype(jnp.bfloat16), W1_ref[...],
                preferred_element_type=jnp.float32)
    x = x + b1_ref[...]
    xb = x.astype(jnp.bfloat16)
    q_ref[0] = jnp.dot(xb, Wq_ref[...], preferred_element_type=jnp.float32)
    kx = jnp.dot(xb, Wk_ref[...], preferred_element_type=jnp.float32)
    vx = jnp.dot(xb, Wv_ref[...], preferred_element_type=jnp.float32)
    kb = lax.bitcast_convert_type(kx.astype(jnp.bfloat16), jnp.uint16)
    vb = lax.bitcast_convert_type(vx.astype(jnp.bfloat16), jnp.uint16)
    packed = kb.astype(jnp.uint32) | (vb.astype(jnp.uint32) << 16)
    kv_ref[0] = lax.bitcast_convert_type(packed, jnp.float32)

    xyzb = xyz_ref[0]
    xyzT = xyzT_ref[0]
    dot = lax.dot_general(xyzb, xyzT, (((1,), (0,)), ((), ())),
                          preferred_element_type=jnp.float32)
    si = jnp.sum(xyzb * xyzb, axis=1, keepdims=True)
    sj = jnp.sum(xyzT * xyzT, axis=0, keepdims=True)
    d = (-2.0 * dot) + si
    d = d + sj

    cols = []
    for _ in range(_K):
        amin = jnp.argmin(d, axis=1).astype(jnp.int32)[:, None]
        cols.append(amin)
        col = lax.broadcasted_iota(jnp.int32, d.shape, 1)
        d = jnp.where(col == amin, jnp.float32(np.inf), d)
    idxblk = jnp.concatenate(cols, axis=1)
    idx_ref[0] = idxblk + b * n_full


def _proj_knn(xyz, xyzT, features, W1, b1, Wq, Wk, Wv):
    B, N, D = features.shape
    P1 = 256
    full = lambda shape: pl.BlockSpec(shape, lambda b, i: (0, 0))
    return pl.pallas_call(
        _proj_knn_body,
        grid=(B, N // P1),
        in_specs=[
            pl.BlockSpec((1, P1, 3), lambda b, i: (b, i, 0)),
            pl.BlockSpec((1, 3, N), lambda b, i: (b, 0, 0)),
            pl.BlockSpec((1, P1, D), lambda b, i: (b, i, 0)),
            full((D, D)),
            full((1, D)),
            full((D, D)),
            full((D, D)),
            full((D, D)),
        ],
        out_specs=[
            pl.BlockSpec((1, P1, D), lambda b, i: (b, i, 0)),
            pl.BlockSpec((1, P1, D), lambda b, i: (b, i, 0)),
            pl.BlockSpec((1, P1, _K), lambda b, i: (b, i, 0)),
        ],
        out_shape=[
            jax.ShapeDtypeStruct((B, N, D), jnp.float32),
            jax.ShapeDtypeStruct((B, N, D), jnp.float32),
            jax.ShapeDtypeStruct((B, N, _K), jnp.int32),
        ],
    )(xyz, xyzT, features, W1, b1, Wq, Wk, Wv)


def _sc_gather(table_kv, table_x, flat_idx):
    TOT = flat_idx.shape[0]
    w_kv = table_kv.shape[1]
    w_x = table_x.shape[1]
    info = plsc.get_sparse_core_info()
    NC, NS = info.num_cores, info.num_subcores
    NW = NC * NS
    per_w = TOT // NW
    CH = 128
    n_ch = per_w // CH
    mesh = plsc.VectorSubcoreMesh(core_axis_name="c", subcore_axis_name="s")

    @functools.partial(
        pl.kernel,
        mesh=mesh,
        out_type=[
            jax.ShapeDtypeStruct((TOT, w_kv), jnp.float32),
            jax.ShapeDtypeStruct((TOT, w_x), jnp.float32),
        ],
        scratch_types=[
            pltpu.VMEM((CH,), jnp.int32),
            pltpu.VMEM((CH,), jnp.int32),
            pltpu.VMEM((CH, w_kv), jnp.float32),
            pltpu.VMEM((CH, w_x), jnp.float32),
            pltpu.VMEM((CH, w_kv), jnp.float32),
            pltpu.VMEM((CH, w_x), jnp.float32),
            pltpu.SemaphoreType.DMA,
            pltpu.SemaphoreType.DMA,
            pltpu.SemaphoreType.DMA,
            pltpu.SemaphoreType.DMA,
        ],
    )
    def gath(kv_hbm, x_hbm, idx_hbm, okv_hbm, ox_hbm, idx_a, idx_b, rkv_a,
             rx_a, rkv_b, rx_b, sem1a, sem2a, sem1b, sem2b):
        wid = lax.axis_index("s") * NC + lax.axis_index("c")
        base = wid * per_w

        def body(j, carry):
            offa = base + (2 * j) * CH
            offb = offa + CH
            pltpu.sync_copy(idx_hbm.at[pl.ds(offa, CH)], idx_a)
            c1a = pltpu.async_copy(kv_hbm.at[idx_a], rkv_a, sem1a)
            c2a = pltpu.async_copy(x_hbm.at[idx_a], rx_a, sem2a)
            pltpu.sync_copy(idx_hbm.at[pl.ds(offb, CH)], idx_b)
            c1b = pltpu.async_copy(kv_hbm.at[idx_b], rkv_b, sem1b)
            c2b = pltpu.async_copy(x_hbm.at[idx_b], rx_b, sem2b)
            c1a.wait()
            c2a.wait()
            pltpu.sync_copy(rkv_a, okv_hbm.at[pl.ds(offa, CH)])
            pltpu.sync_copy(rx_a, ox_hbm.at[pl.ds(offa, CH)])
            c1b.wait()
            c2b.wait()
            pltpu.sync_copy(rkv_b, okv_hbm.at[pl.ds(offb, CH)])
            pltpu.sync_copy(rx_b, ox_hbm.at[pl.ds(offb, CH)])
            return carry

        lax.fori_loop(0, n_ch // 2, body, 0)

    return gath(table_kv, table_x, flat_idx)


def _attn_body(q_ref, feat_ref, xyzp_ref, gkv_ref, gx_ref, Wd1_ref, bd1_ref,
               Wd2_ref, bd2_ref, Wg1_ref, bg1_ref, Wg2_ref, bg2_ref, W2_ref,
               b2_ref, nfi_ref, atti_ref, nf_ref, attn_ref):
    P2, K, D = gkv_ref.shape
    gkv = gkv_ref[...]
    u = lax.bitcast_convert_type(gkv, jnp.int32)
    gk = lax.bitcast_convert_type(u << 16, jnp.float32).reshape(P2 * K, D)
    gv = lax.bitcast_convert_type(u & jnp.int32(-65536), jnp.float32)
    gx = gx_ref[:, :, :16]

    xyzp = xyzp_ref[...]
    delta = (xyzp[:, None, :] - gx).reshape(P2 * K, 16)
    h1 = jnp.dot(delta.astype(jnp.bfloat16), Wd1_ref[...],
                 preferred_element_type=jnp.float32)
    h1 = jnp.maximum(h1 + bd1_ref[...], 0.0)
    pos = jnp.dot(h1.astype(jnp.bfloat16), Wd2_ref[...],
                  preferred_element_type=jnp.float32)
    pos = pos + bd2_ref[...]

    q = q_ref[...]
    qb = jnp.broadcast_to(q[:, None, :], (P2, K, D)).reshape(P2 * K, D)
    pre = qb - gk + pos
    h2 = jnp.dot(pre.astype(jnp.bfloat16), Wg1_ref[...],
                 preferred_element_type=jnp.float32)
    h2 = jnp.maximum(h2 + bg1_ref[...], 0.0)
    logits = jnp.dot(h2.astype(jnp.bfloat16), Wg2_ref[...],
                     preferred_element_type=jnp.float32)
    logits = (logits + bg2_ref[...]) * (1.0 / 16.0)

    l3 = logits.reshape(P2, K, D)
    m = jnp.max(l3, axis=1, keepdims=True)
    e = jnp.exp(l3 - m)
    s = jnp.sum(e, axis=1, keepdims=True)
    attn = e / s
    attn_ref[...] = attn

    wsum = jnp.sum(attn * (gv + pos.reshape(P2, K, D)), axis=1)
    nf = jnp.dot(wsum.astype(jnp.bfloat16), W2_ref[...],
                 preferred_element_type=jnp.float32)
    nf_ref[...] = nf + b2_ref[...] + feat_ref[...]


def _attn(q, feat, xyzp, gkv, gx, Wd1p, bd1, Wd2, bd2, Wg1, bg1, Wg2, bg2,
          W2, b2, nf_buf, attn_buf, blk_off):
    BN, D = q.shape
    BNF = nf_buf.shape[0]
    P2 = 256
    full = lambda shape: pl.BlockSpec(shape, lambda i: (0, 0))
    return pl.pallas_call(
        _attn_body,
        grid=(BN // P2,),
        in_specs=[
            pl.BlockSpec((P2, D), lambda i: (i, 0)),
            pl.BlockSpec((P2, D), lambda i: (i, 0)),
            pl.BlockSpec((P2, 16), lambda i: (i, 0)),
            pl.BlockSpec((P2, _K, D), lambda i: (i, 0, 0)),
            pl.BlockSpec((P2, _K, 128), lambda i: (i, 0, 0)),
            full((16, D)),
            full((1, D)),
            full((D, D)),
            full((1, D)),
            full((D, D)),
            full((1, D)),
            full((D, D)),
            full((1, D)),
            full((D, D)),
            full((1, D)),
            pl.BlockSpec(memory_space=pl.ANY),
            pl.BlockSpec(memory_space=pl.ANY),
        ],
        out_specs=[
            pl.BlockSpec((P2, D), lambda i: (i + blk_off, 0)),
            pl.BlockSpec((P2, _K, D), lambda i: (i + blk_off, 0, 0)),
        ],
        out_shape=[
            jax.ShapeDtypeStruct((BNF, D), jnp.float32),
            jax.ShapeDtypeStruct((BNF, _K, D), jnp.float32),
        ],
        input_output_aliases={15: 0, 16: 1},
    )(q, feat, xyzp, gkv, gx, Wd1p, bd1, Wd2, bd2, Wg1, bg1, Wg2, bg2, W2,
      b2, nf_buf, attn_buf)


def _half_pipeline(xyz, features, nf_buf, attn_buf, blk_off, W1, b1, Wd1p,
                   bd1, Wd2, bd2, Wg1, bg1, Wg2, bg2, W2, b2, Wq, Wk, Wv):
    B, N, _ = xyz.shape
    D = features.shape[-1]
    BN = B * N

    xyzT = jnp.swapaxes(xyz, 1, 2)
    q, kv, gidx = _proj_knn(
        xyz, xyzT, features, W1, b1, Wq, Wk, Wv)

    xyzp = jnp.concatenate(
        [xyz, jnp.zeros((B, N, 13), jnp.float32)], axis=-1)
    xyzp128 = jnp.concatenate(
        [xyzp, jnp.zeros((B, N, 112), jnp.float32)], axis=-1)
    flat_idx = gidx.reshape(BN * _K)

    gkv, gx = _sc_gather(kv.reshape(BN, D), xyzp128.reshape(BN, 128),
                         flat_idx)
    gkv = gkv.reshape(BN, _K, D)
    gx = gx.reshape(BN, _K, 128)

    return _attn(
        q.reshape(BN, D), features.reshape(BN, D), xyzp.reshape(BN, 16),
        gkv, gx, Wd1p, bd1, Wd2, bd2, Wg1, bg1, Wg2, bg2, W2, b2,
        nf_buf, attn_buf, blk_off)


def kernel(xyz, features, W1, b1, W2, b2, Wd1, bd1, Wd2, bd2, Wg1, bg1,
           Wg2, bg2, Wq, Wk, Wv):
    B, N, _ = xyz.shape
    D = features.shape[-1]

    Wd1p = jnp.concatenate(
        [Wd1, jnp.zeros((13, D), jnp.float32)], axis=0)
    bf = jnp.bfloat16
    ws = (W1.astype(bf), b1.reshape(1, D), Wd1p.astype(bf),
          bd1.reshape(1, D), Wd2.astype(bf), bd2.reshape(1, D),
          Wg1.astype(bf), bg1.reshape(1, D), Wg2.astype(bf),
          bg2.reshape(1, D), W2.astype(bf), b2.reshape(1, D),
          Wq.astype(bf), Wk.astype(bf), Wv.astype(bf))

    h = B // 2
    BN = B * N
    P2 = 256
    nf_buf = jnp.zeros((BN, D), jnp.float32)
    attn_buf = jnp.zeros((BN, _K, D), jnp.float32)
    for hi, s in enumerate((slice(0, h), slice(h, B))):
        nf_buf, attn_buf = _half_pipeline(
            xyz[s], features[s], nf_buf, attn_buf,
            hi * (h * N // P2), *ws)
    return (nf_buf.reshape(B, N, D), attn_buf.reshape(B, N, _K, D))
